# baseline (speedup 1.0000x reference)
import jax
import jax.numpy as jnp
from jax import lax
from jax.experimental import pallas as pl
from jax.experimental.pallas import tpu as pltpu
from jax.experimental.pallas import tpu_sc as plsc

N_NODES = 10000
D_FEAT = 128
C_OUT = 40
C_PAD = 48
E_ORIG = 320000

NC, NS, L = 2, 16, 16
NW = NC * NS

N_PAD = 10240
IDX_W = 128
ROWS_E = E_ORIG // IDX_W
RPT = ROWS_E // NW
N_XTRA = ROWS_E - RPT * NW
XBASE = RPT * NW
BLK = 1024
G = 3
NPAIR = RPT // (2 * G)


def _deg_body(ei_hbm, hist_out, dstbuf, histbuf):
    c = lax.axis_index("c")
    s = lax.axis_index("s")
    wid = s * NC + c
    zeros = jnp.zeros((L,), jnp.float32)

    def zbody(i, carry):
        histbuf[i, pl.ds(0, L)] = zeros
        return carry

    lax.fori_loop(0, N_PAD // L, zbody, 0)
    pltpu.sync_copy(ei_hbm.at[1, pl.ds(wid * RPT, RPT)], dstbuf)
    ones = jnp.ones((L,), jnp.float32)
    gpr = IDX_W // L

    def count_row(r, j):
        idx = dstbuf[r, pl.ds(j * L, L)]
        plsc.addupdate_scatter(histbuf, [idx >> 4, idx & 15], ones)

    def body(i, carry):
        count_row(i // gpr, i % gpr)
        return carry

    lax.fori_loop(0, RPT * gpr, body, 0)

    @pl.when(wid < N_XTRA)
    def _():
        pltpu.sync_copy(ei_hbm.at[1, pl.ds(XBASE + wid, 1)],
                        dstbuf.at[pl.ds(0, 1)])

        def xbody(j, carry):
            count_row(0, j)
            return carry

        lax.fori_loop(0, gpr, xbody, 0)

    pltpu.sync_copy(histbuf, hist_out.at[wid])


def _deg_call(ei3):
    mesh = plsc.VectorSubcoreMesh(core_axis_name="c", subcore_axis_name="s")
    return pl.kernel(
        _deg_body,
        out_type=jax.ShapeDtypeStruct((NW, N_PAD // L, L), jnp.float32),
        mesh=mesh,
        scratch_types=[
            pltpu.VMEM((RPT, IDX_W), jnp.int32),
            pltpu.VMEM((N_PAD // L, L), jnp.float32),
        ],
        compiler_params=pltpu.CompilerParams(
            needs_layout_passes=False, use_tc_tiling_on_sc=False),
    )(ei3)


RPS = N_PAD // NW
R16 = RPS // L


def _rsqrt_vec(deg):
    i = plsc.bitcast(deg, jnp.int32)
    y = plsc.bitcast(jnp.int32(0x5F3759DF) - (i >> 1), jnp.float32)
    for _ in range(3):
        y = y * (1.5 - 0.5 * deg * y * y)
    return y


def _g0sc_body(z_hbm, hist_hbm, g0_hbm, dinv_hbm, d2_hbm,
               zv, hv, g0v, dv, d2v, sem):
    c = lax.axis_index("c")
    s = lax.axis_index("s")
    wid = s * NC + c
    base = wid * RPS
    cp1 = pltpu.async_copy(hist_hbm.at[:, pl.ds(wid * R16, R16), :], hv, sem)
    cp2 = pltpu.async_copy(z_hbm.at[pl.ds(base, RPS)], zv, sem)
    cp1.wait()
    cp2.wait()

    def dbody(r2, carry):
        acc = hv[0, r2, pl.ds(0, L)]
        for w in range(1, NW):
            acc = acc + hv[w, r2, pl.ds(0, L)]
        deg = acc + 1.0
        y = _rsqrt_vec(deg)
        dv[pl.ds(r2 * L, L)] = y
        d2v[pl.ds(r2 * L, L)] = y * y
        return carry

    lax.fori_loop(0, R16, dbody, 0)

    def rbody(r4, carry):
        for u in range(4):
            r = r4 * 4 + u
            dd = plsc.load_gather(dv, [jnp.full((L,), r, jnp.int32)])
            for j in range(C_PAD // L):
                g0v[r, pl.ds(j * L, L)] = zv[r, pl.ds(j * L, L)] * dd
        return carry

    lax.fori_loop(0, RPS // 4, rbody, 0)
    pltpu.sync_copy(g0v, g0_hbm.at[pl.ds(base, RPS)])
    pltpu.sync_copy(dv, dinv_hbm.at[pl.ds(base, RPS)])
    pltpu.sync_copy(d2v, d2_hbm.at[pl.ds(base, RPS)])


def _g0sc_call(z, hist3):
    mesh = plsc.VectorSubcoreMesh(core_axis_name="c", subcore_axis_name="s")
    return pl.kernel(
        _g0sc_body,
        out_type=[
            jax.ShapeDtypeStruct((N_PAD, C_PAD), jnp.float32),
            jax.ShapeDtypeStruct((N_PAD,), jnp.float32),
            jax.ShapeDtypeStruct((N_PAD,), jnp.float32),
        ],
        mesh=mesh,
        scratch_types=[
            pltpu.VMEM((RPS, C_PAD), jnp.float32),
            pltpu.VMEM((NW, R16, L), jnp.float32),
            pltpu.VMEM((RPS, C_PAD), jnp.float32),
            pltpu.VMEM((RPS,), jnp.float32),
            pltpu.VMEM((RPS,), jnp.float32),
            pltpu.SemaphoreType.DMA,
        ],
        compiler_params=pltpu.CompilerParams(
            needs_layout_passes=False, use_tc_tiling_on_sc=False),
    )(z, hist3)


def _scalesc_body(g0_hbm, parts_hbm, d2_hbm, g1_hbm,
                  g0v, pav, pbv, d2v, g1v, sem):
    c = lax.axis_index("c")
    s = lax.axis_index("s")
    wid = s * NC + c
    base = wid * RPS
    cps = [
        pltpu.async_copy(g0_hbm.at[pl.ds(base, RPS)], g0v, sem),
        pltpu.async_copy(parts_hbm.at[0, pl.ds(base, RPS)], pav, sem),
        pltpu.async_copy(parts_hbm.at[1, pl.ds(base, RPS)], pbv, sem),
        pltpu.async_copy(d2_hbm.at[pl.ds(base, RPS)], d2v, sem),
    ]
    for cp in cps:
        cp.wait()

    def rbody(r4, carry):
        for u in range(4):
            r = r4 * 4 + u
            dd = plsc.load_gather(d2v, [jnp.full((L,), r, jnp.int32)])
            for j in range(C_PAD // L):
                sl = pl.ds(j * L, L)
                g1v[r, sl] = (g0v[r, sl] + pav[r, sl] + pbv[r, sl]) * dd
        return carry

    lax.fori_loop(0, RPS // 4, rbody, 0)
    pltpu.sync_copy(g1v, g1_hbm.at[pl.ds(base, RPS)])


def _scalesc_call(g0, parts, d2):
    mesh = plsc.VectorSubcoreMesh(core_axis_name="c", subcore_axis_name="s")
    return pl.kernel(
        _scalesc_body,
        out_type=jax.ShapeDtypeStruct((N_PAD, C_PAD), jnp.float32),
        mesh=mesh,
        scratch_types=[
            pltpu.VMEM((RPS, C_PAD), jnp.float32),
            pltpu.VMEM((RPS, C_PAD), jnp.float32),
            pltpu.VMEM((RPS, C_PAD), jnp.float32),
            pltpu.VMEM((RPS,), jnp.float32),
            pltpu.VMEM((RPS, C_PAD), jnp.float32),
            pltpu.SemaphoreType.DMA,
        ],
        compiler_params=pltpu.CompilerParams(
            needs_layout_passes=False, use_tc_tiling_on_sc=False),
    )(g0, parts, d2)


def _prop_body(g_hbm, ei_hbm, zrow_hbm, out_hbm, srcbuf, dstbuf, *rest):
    bufs_a = rest[:G]
    bufs_b = rest[G:2 * G]
    semga, semgb, semsa, semsb, acc = rest[2 * G:]
    c = lax.axis_index("c")
    s = lax.axis_index("s")
    wid = s * NC + c
    rps = N_PAD // NS

    def zcopy(i, carry):
        pltpu.sync_copy(zrow_hbm, acc.at[pl.ds(s * rps + i * IDX_W, IDX_W)])
        return carry

    with jax.named_scope("ldidx"):
        cps = pltpu.async_copy(ei_hbm.at[0, pl.ds(wid * RPT, RPT)], srcbuf,
                               semga)
        cpd = pltpu.async_copy(ei_hbm.at[1, pl.ds(wid * RPT, RPT)], dstbuf,
                               semgb)
    with jax.named_scope("acc_zero"):
        lax.fori_loop(0, rps // IDX_W, zcopy, 0)
    cps.wait()
    cpd.wait()

    def gath(j, buf, sem):
        pltpu.async_copy(g_hbm.at[srcbuf.at[j]], buf, sem)

    def gwait(buf, sem):
        pltpu.make_async_copy(g_hbm.at[srcbuf.at[0]], buf, sem).wait()

    def scat(j, buf, sem):
        pltpu.async_copy(buf, acc.at[dstbuf.at[j]], sem, add=True)

    def swait(buf, sem):
        pltpu.make_async_copy(buf, acc.at[dstbuf.at[0]], sem).wait()

    for k in range(G):
        gath(k, bufs_a[k], semga)
    plsc.subcore_barrier()

    def pbody(i, carry):
        ja = (2 * i) * G
        jb = ja + G
        for k in range(G):
            gath(jb + k, bufs_b[k], semgb)
        for k in range(G):
            gwait(bufs_a[k], semga)
        for k in range(G):
            scat(ja + k, bufs_a[k], semsa)

        @pl.when(i < NPAIR - 1)
        def _():
            for k in range(G):
                swait(bufs_a[k], semsa)
            for k in range(G):
                gath(jb + G + k, bufs_a[k], semga)

        for k in range(G):
            gwait(bufs_b[k], semgb)
        for k in range(G):
            scat(jb + k, bufs_b[k], semsb)
        for k in range(G):
            swait(bufs_b[k], semsb)
        return carry

    with jax.named_scope("edges"):
        lax.fori_loop(0, NPAIR, pbody, 0)
        for k in range(G):
            swait(bufs_a[k], semsa)

        @pl.when(wid < N_XTRA)
        def _():
            pltpu.sync_copy(ei_hbm.at[0, pl.ds(XBASE + wid, 1)],
                            srcbuf.at[pl.ds(0, 1)])
            pltpu.sync_copy(ei_hbm.at[1, pl.ds(XBASE + wid, 1)],
                            dstbuf.at[pl.ds(0, 1)])
            gath(0, bufs_a[0], semga)
            gwait(bufs_a[0], semga)
            scat(0, bufs_a[0], semsa)
            swait(bufs_a[0], semsa)

    plsc.subcore_barrier()
    with jax.named_scope("wb"):
        pltpu.sync_copy(acc.at[pl.ds(s * rps, rps)],
                        out_hbm.at[c, pl.ds(s * rps, rps)])


def _prop_call(g, ei3, zrow):
    mesh = plsc.VectorSubcoreMesh(core_axis_name="c", subcore_axis_name="s")
    return pl.kernel(
        _prop_body,
        out_type=jax.ShapeDtypeStruct((NC, N_PAD, C_PAD), jnp.float32),
        mesh=mesh,
        scratch_types=(
            [pltpu.VMEM((RPT, IDX_W), jnp.int32),
             pltpu.VMEM((RPT, IDX_W), jnp.int32)]
            + [pltpu.VMEM((IDX_W, C_PAD), jnp.float32)] * (2 * G)
            + [pltpu.SemaphoreType.DMA] * 4
            + [pltpu.VMEM_SHARED((N_PAD, C_PAD), jnp.float32)]
        ),
        compiler_params=pltpu.CompilerParams(
            needs_layout_passes=False, use_tc_tiling_on_sc=False),
    )(g, ei3, zrow)


def _mm_body(x_ref, w_ref, z_ref):
    z_ref[...] = jnp.dot(x_ref[...], w_ref[...],
                         preferred_element_type=jnp.float32)


def _mm_call(x_pad, w_pad):
    return pl.pallas_call(
        _mm_body,
        grid=(N_PAD // BLK,),
        in_specs=[
            pl.BlockSpec((BLK, D_FEAT), lambda i: (i, 0)),
            pl.BlockSpec((D_FEAT, C_PAD), lambda i: (0, 0)),
        ],
        out_specs=pl.BlockSpec((BLK, C_PAD), lambda i: (i, 0)),
        out_shape=jax.ShapeDtypeStruct((N_PAD, C_PAD), jnp.float32),
    )(x_pad, w_pad)


def _final_body(g1_ref, q_ref, dinv_ref, b_ref, out_ref):
    t = g1_ref[...] + q_ref[0] + q_ref[1]
    logits = (t * dinv_ref[...][:, None])[:, :C_OUT] + b_ref[...][None, :]
    m = jnp.max(logits, axis=1, keepdims=True)
    e = jnp.exp(logits - m)
    out_ref[...] = logits - m - jnp.log(jnp.sum(e, axis=1, keepdims=True))


def _final_call(g1, parts, dinv, b):
    return pl.pallas_call(
        _final_body,
        grid=(N_PAD // BLK,),
        in_specs=[
            pl.BlockSpec((BLK, C_PAD), lambda i: (i, 0)),
            pl.BlockSpec((NC, BLK, C_PAD), lambda i: (0, i, 0)),
            pl.BlockSpec((BLK,), lambda i: (i,)),
            pl.BlockSpec((C_OUT,), lambda i: (0,)),
        ],
        out_specs=pl.BlockSpec((BLK, C_OUT), lambda i: (i, 0)),
        out_shape=jax.ShapeDtypeStruct((N_PAD, C_OUT), jnp.float32),
    )(g1, parts, dinv, b)


def kernel(x, edge_index, W, b):
    ei3 = edge_index.astype(jnp.int32).reshape(2, ROWS_E, IDX_W)
    x_pad = jnp.pad(x, ((0, N_PAD - N_NODES), (0, 0)))
    w_pad = jnp.pad(W, ((0, 0), (0, C_PAD - C_OUT)))

    zrow = jnp.zeros((IDX_W, C_PAD), jnp.float32)

    z = _mm_call(x_pad, w_pad)
    hist3 = _deg_call(ei3)
    g0, dinv, d2 = _g0sc_call(z, hist3)
    s0 = _prop_call(g0, ei3, zrow)
    g1 = _scalesc_call(g0, s0, d2)
    s1 = _prop_call(g1, ei3, zrow)
    out = _final_call(g1, s1, dinv, b)
    return out[:N_NODES]


# speedup vs baseline: 1.0864x; 1.0864x over previous
import jax
import jax.numpy as jnp
from jax import lax
from jax.experimental import pallas as pl
from jax.experimental.pallas import tpu as pltpu
from jax.experimental.pallas import tpu_sc as plsc

N_NODES = 10000
D_FEAT = 128
C_OUT = 40
C_PAD = 48
E_ORIG = 320000

NC, NS, L = 2, 16, 16
NW = NC * NS

N_PAD = 10240
IDX_W = 128
ROWS_E = E_ORIG // IDX_W
RPT = ROWS_E // NW
N_XTRA = ROWS_E - RPT * NW
XBASE = RPT * NW
BLK = 1024
G = 3
NPAIR = RPT // (2 * G)


def _deg_body(ei_hbm, hist_out, dstbuf, histbuf):
    c = lax.axis_index("c")
    s = lax.axis_index("s")
    wid = s * NC + c
    zeros = jnp.zeros((L,), jnp.float32)

    def zbody(i, carry):
        histbuf[i, pl.ds(0, L)] = zeros
        return carry

    lax.fori_loop(0, N_PAD // L, zbody, 0)
    pltpu.sync_copy(ei_hbm.at[1, pl.ds(wid * RPT, RPT)], dstbuf)
    ones = jnp.ones((L,), jnp.float32)
    gpr = IDX_W // L

    def count_row(r, j):
        idx = dstbuf[r, pl.ds(j * L, L)]
        plsc.addupdate_scatter(histbuf, [idx >> 4, idx & 15], ones)

    def body(i, carry):
        count_row(i // gpr, i % gpr)
        return carry

    lax.fori_loop(0, RPT * gpr, body, 0)

    @pl.when(wid < N_XTRA)
    def _():
        pltpu.sync_copy(ei_hbm.at[1, pl.ds(XBASE + wid, 1)],
                        dstbuf.at[pl.ds(0, 1)])

        def xbody(j, carry):
            count_row(0, j)
            return carry

        lax.fori_loop(0, gpr, xbody, 0)

    pltpu.sync_copy(histbuf, hist_out.at[wid])


def _deg_call(ei3):
    mesh = plsc.VectorSubcoreMesh(core_axis_name="c", subcore_axis_name="s")
    return pl.kernel(
        _deg_body,
        out_type=jax.ShapeDtypeStruct((NW, N_PAD // L, L), jnp.float32),
        mesh=mesh,
        scratch_types=[
            pltpu.VMEM((RPT, IDX_W), jnp.int32),
            pltpu.VMEM((N_PAD // L, L), jnp.float32),
        ],
        compiler_params=pltpu.CompilerParams(
            needs_layout_passes=False, use_tc_tiling_on_sc=False),
    )(ei3)


RPS = N_PAD // NW
R16 = RPS // L


def _rsqrt_vec(deg):
    i = plsc.bitcast(deg, jnp.int32)
    y = plsc.bitcast(jnp.int32(0x5F3759DF) - (i >> 1), jnp.float32)
    for _ in range(3):
        y = y * (1.5 - 0.5 * deg * y * y)
    return y


def _g0sc_body(z_hbm, hist_hbm, g0_hbm, dinv_hbm, d2_hbm,
               zv, hv, g0v, dv, d2v, sem):
    c = lax.axis_index("c")
    s = lax.axis_index("s")
    wid = s * NC + c
    base = wid * RPS
    cp1 = pltpu.async_copy(hist_hbm.at[:, pl.ds(wid * R16, R16), :], hv, sem)
    cp2 = pltpu.async_copy(z_hbm.at[pl.ds(base, RPS)], zv, sem)
    cp1.wait()
    cp2.wait()

    def dbody(r2, carry):
        acc = hv[0, r2, pl.ds(0, L)]
        for w in range(1, NW):
            acc = acc + hv[w, r2, pl.ds(0, L)]
        deg = acc + 1.0
        y = _rsqrt_vec(deg)
        dv[pl.ds(r2 * L, L)] = y
        d2v[pl.ds(r2 * L, L)] = y * y
        return carry

    lax.fori_loop(0, R16, dbody, 0)

    def rbody(r4, carry):
        for u in range(4):
            r = r4 * 4 + u
            dd = plsc.load_gather(dv, [jnp.full((L,), r, jnp.int32)])
            for j in range(C_PAD // L):
                g0v[r, pl.ds(j * L, L)] = zv[r, pl.ds(j * L, L)] * dd
        return carry

    lax.fori_loop(0, RPS // 4, rbody, 0)
    pltpu.sync_copy(g0v, g0_hbm.at[pl.ds(base, RPS)])
    pltpu.sync_copy(dv, dinv_hbm.at[pl.ds(base, RPS)])
    pltpu.sync_copy(d2v, d2_hbm.at[pl.ds(base, RPS)])


def _g0sc_call(z, hist3):
    mesh = plsc.VectorSubcoreMesh(core_axis_name="c", subcore_axis_name="s")
    return pl.kernel(
        _g0sc_body,
        out_type=[
            jax.ShapeDtypeStruct((N_PAD, C_PAD), jnp.float32),
            jax.ShapeDtypeStruct((N_PAD,), jnp.float32),
            jax.ShapeDtypeStruct((N_PAD,), jnp.float32),
        ],
        mesh=mesh,
        scratch_types=[
            pltpu.VMEM((RPS, C_PAD), jnp.float32),
            pltpu.VMEM((NW, R16, L), jnp.float32),
            pltpu.VMEM((RPS, C_PAD), jnp.float32),
            pltpu.VMEM((RPS,), jnp.float32),
            pltpu.VMEM((RPS,), jnp.float32),
            pltpu.SemaphoreType.DMA,
        ],
        compiler_params=pltpu.CompilerParams(
            needs_layout_passes=False, use_tc_tiling_on_sc=False),
    )(z, hist3)


def _scalesc_body(g0_hbm, parts_hbm, d2_hbm, g1_hbm,
                  g0v, pav, pbv, d2v, g1v, sem):
    c = lax.axis_index("c")
    s = lax.axis_index("s")
    wid = s * NC + c
    base = wid * RPS
    cps = [
        pltpu.async_copy(g0_hbm.at[pl.ds(base, RPS)], g0v, sem),
        pltpu.async_copy(parts_hbm.at[0, pl.ds(base, RPS)], pav, sem),
        pltpu.async_copy(parts_hbm.at[1, pl.ds(base, RPS)], pbv, sem),
        pltpu.async_copy(d2_hbm.at[pl.ds(base, RPS)], d2v, sem),
    ]
    for cp in cps:
        cp.wait()

    def rbody(r4, carry):
        for u in range(4):
            r = r4 * 4 + u
            dd = plsc.load_gather(d2v, [jnp.full((L,), r, jnp.int32)])
            for j in range(C_PAD // L):
                sl = pl.ds(j * L, L)
                g1v[r, sl] = (g0v[r, sl] + pav[r, sl] + pbv[r, sl]) * dd
        return carry

    lax.fori_loop(0, RPS // 4, rbody, 0)
    pltpu.sync_copy(g1v, g1_hbm.at[pl.ds(base, RPS)])


def _scalesc_call(g0, parts, d2):
    mesh = plsc.VectorSubcoreMesh(core_axis_name="c", subcore_axis_name="s")
    return pl.kernel(
        _scalesc_body,
        out_type=jax.ShapeDtypeStruct((N_PAD, C_PAD), jnp.float32),
        mesh=mesh,
        scratch_types=[
            pltpu.VMEM((RPS, C_PAD), jnp.float32),
            pltpu.VMEM((RPS, C_PAD), jnp.float32),
            pltpu.VMEM((RPS, C_PAD), jnp.float32),
            pltpu.VMEM((RPS,), jnp.float32),
            pltpu.VMEM((RPS, C_PAD), jnp.float32),
            pltpu.SemaphoreType.DMA,
        ],
        compiler_params=pltpu.CompilerParams(
            needs_layout_passes=False, use_tc_tiling_on_sc=False),
    )(g0, parts, d2)


def _prop_body(g_hbm, ei_hbm, out_hbm, srcbuf, dstbuf, *rest):
    bufs_a = rest[:G]
    bufs_b = rest[G:2 * G]
    semga, semgb, semsa, semsb, acc = rest[2 * G:]
    c = lax.axis_index("c")
    s = lax.axis_index("s")
    wid = s * NC + c
    rps = N_PAD // NS

    with jax.named_scope("ldidx"):
        cps = pltpu.async_copy(ei_hbm.at[0, pl.ds(wid * RPT, RPT)], srcbuf,
                               semga)
        cpd = pltpu.async_copy(ei_hbm.at[1, pl.ds(wid * RPT, RPT)], dstbuf,
                               semgb)
    zeros = jnp.zeros((L,), jnp.float32)

    def zbody(i, carry):
        r = i // (C_PAD // L)
        j = i % (C_PAD // L)
        bufs_a[0][r, pl.ds(j * L, L)] = zeros
        return carry

    lax.fori_loop(0, IDX_W * (C_PAD // L), zbody, 0)

    def zcopy(i, carry):
        pltpu.sync_copy(bufs_a[0], acc.at[pl.ds(s * rps + i * IDX_W, IDX_W)])
        return carry

    with jax.named_scope("acc_zero"):
        lax.fori_loop(0, rps // IDX_W, zcopy, 0)
    cps.wait()
    cpd.wait()

    def gath(j, buf, sem):
        pltpu.async_copy(g_hbm.at[srcbuf.at[j]], buf, sem)

    def gwait(buf, sem):
        pltpu.make_async_copy(g_hbm.at[srcbuf.at[0]], buf, sem).wait()

    def scat(j, buf, sem):
        pltpu.async_copy(buf, acc.at[dstbuf.at[j]], sem, add=True)

    def swait(buf, sem):
        pltpu.make_async_copy(buf, acc.at[dstbuf.at[0]], sem).wait()

    for k in range(G):
        gath(k, bufs_a[k], semga)
    plsc.subcore_barrier()

    def pbody(i, carry):
        ja = (2 * i) * G
        jb = ja + G
        for k in range(G):
            gath(jb + k, bufs_b[k], semgb)
        for k in range(G):
            gwait(bufs_a[k], semga)
        for k in range(G):
            scat(ja + k, bufs_a[k], semsa)

        @pl.when(i < NPAIR - 1)
        def _():
            for k in range(G):
                swait(bufs_a[k], semsa)
            for k in range(G):
                gath(jb + G + k, bufs_a[k], semga)

        for k in range(G):
            gwait(bufs_b[k], semgb)
        for k in range(G):
            scat(jb + k, bufs_b[k], semsb)
        for k in range(G):
            swait(bufs_b[k], semsb)
        return carry

    with jax.named_scope("edges"):
        lax.fori_loop(0, NPAIR, pbody, 0)
        for k in range(G):
            swait(bufs_a[k], semsa)

        @pl.when(wid < N_XTRA)
        def _():
            pltpu.sync_copy(ei_hbm.at[0, pl.ds(XBASE + wid, 1)],
                            srcbuf.at[pl.ds(0, 1)])
            pltpu.sync_copy(ei_hbm.at[1, pl.ds(XBASE + wid, 1)],
                            dstbuf.at[pl.ds(0, 1)])
            gath(0, bufs_a[0], semga)
            gwait(bufs_a[0], semga)
            scat(0, bufs_a[0], semsa)
            swait(bufs_a[0], semsa)

    plsc.subcore_barrier()
    with jax.named_scope("wb"):
        pltpu.sync_copy(acc.at[pl.ds(s * rps, rps)],
                        out_hbm.at[c, pl.ds(s * rps, rps)])


def _prop_call(g, ei3):
    mesh = plsc.VectorSubcoreMesh(core_axis_name="c", subcore_axis_name="s")
    return pl.kernel(
        _prop_body,
        out_type=jax.ShapeDtypeStruct((NC, N_PAD, C_PAD), jnp.float32),
        mesh=mesh,
        scratch_types=(
            [pltpu.VMEM((RPT, IDX_W), jnp.int32),
             pltpu.VMEM((RPT, IDX_W), jnp.int32)]
            + [pltpu.VMEM((IDX_W, C_PAD), jnp.float32)] * (2 * G)
            + [pltpu.SemaphoreType.DMA] * 4
            + [pltpu.VMEM_SHARED((N_PAD, C_PAD), jnp.float32)]
        ),
        compiler_params=pltpu.CompilerParams(
            needs_layout_passes=False, use_tc_tiling_on_sc=False),
    )(g, ei3)


def _mm_body(x_ref, w_ref, z_ref):
    z_ref[...] = jnp.dot(x_ref[...], w_ref[...],
                         preferred_element_type=jnp.float32)


def _mm_call(x_pad, w_pad):
    return pl.pallas_call(
        _mm_body,
        grid=(N_PAD // BLK,),
        in_specs=[
            pl.BlockSpec((BLK, D_FEAT), lambda i: (i, 0)),
            pl.BlockSpec((D_FEAT, C_PAD), lambda i: (0, 0)),
        ],
        out_specs=pl.BlockSpec((BLK, C_PAD), lambda i: (i, 0)),
        out_shape=jax.ShapeDtypeStruct((N_PAD, C_PAD), jnp.float32),
    )(x_pad, w_pad)


def _final_body(g1_ref, q_ref, dinv_ref, b_ref, out_ref):
    t = g1_ref[...] + q_ref[0] + q_ref[1]
    logits = (t * dinv_ref[...][:, None])[:, :C_OUT] + b_ref[...][None, :]
    m = jnp.max(logits, axis=1, keepdims=True)
    e = jnp.exp(logits - m)
    out_ref[...] = logits - m - jnp.log(jnp.sum(e, axis=1, keepdims=True))


def _final_call(g1, parts, dinv, b):
    return pl.pallas_call(
        _final_body,
        grid=(N_PAD // BLK,),
        in_specs=[
            pl.BlockSpec((BLK, C_PAD), lambda i: (i, 0)),
            pl.BlockSpec((NC, BLK, C_PAD), lambda i: (0, i, 0)),
            pl.BlockSpec((BLK,), lambda i: (i,)),
            pl.BlockSpec((C_OUT,), lambda i: (0,)),
        ],
        out_specs=pl.BlockSpec((BLK, C_OUT), lambda i: (i, 0)),
        out_shape=jax.ShapeDtypeStruct((N_PAD, C_OUT), jnp.float32),
    )(g1, parts, dinv, b)


def kernel(x, edge_index, W, b):
    ei3 = edge_index.astype(jnp.int32).reshape(2, ROWS_E, IDX_W)
    x_pad = jnp.pad(x, ((0, N_PAD - N_NODES), (0, 0)))
    w_pad = jnp.pad(W, ((0, 0), (0, C_PAD - C_OUT)))

    z = _mm_call(x_pad, w_pad)
    hist3 = _deg_call(ei3)
    g0, dinv, d2 = _g0sc_call(z, hist3)
    s0 = _prop_call(g0, ei3)
    g1 = _scalesc_call(g0, s0, d2)
    s1 = _prop_call(g1, ei3)
    out = _final_call(g1, s1, dinv, b)
    return out[:N_NODES]


# speedup vs baseline: 1.1144x; 1.0258x over previous
import jax
import jax.numpy as jnp
from jax import lax
from jax.experimental import pallas as pl
from jax.experimental.pallas import tpu as pltpu
from jax.experimental.pallas import tpu_sc as plsc

N_NODES = 10000
D_FEAT = 128
C_OUT = 40
C_PAD = 48
E_ORIG = 320000

NC, NS, L = 2, 16, 16
NW = NC * NS

IDX_W = 128
ROWS_E = E_ORIG // IDX_W
RPT = ROWS_E // NW
N_XTRA = ROWS_E - RPT * NW
XBASE = RPT * NW
BLK = 1000
G = 3
NPAIR = RPT // (2 * G)


def _deg_body(ei_hbm, hist_out, dstbuf, histbuf):
    c = lax.axis_index("c")
    s = lax.axis_index("s")
    wid = s * NC + c
    zeros = jnp.zeros((L,), jnp.float32)

    def zbody(i, carry):
        histbuf[i, pl.ds(0, L)] = zeros
        return carry

    lax.fori_loop(0, N_NODES // L, zbody, 0)
    pltpu.sync_copy(ei_hbm.at[1, pl.ds(wid * RPT, RPT)], dstbuf)
    ones = jnp.ones((L,), jnp.float32)
    gpr = IDX_W // L

    def count_row(r, j):
        idx = dstbuf[r, pl.ds(j * L, L)]
        plsc.addupdate_scatter(histbuf, [idx >> 4, idx & 15], ones)

    def body(i, carry):
        count_row(i // gpr, i % gpr)
        return carry

    lax.fori_loop(0, RPT * gpr, body, 0)

    @pl.when(wid < N_XTRA)
    def _():
        pltpu.sync_copy(ei_hbm.at[1, pl.ds(XBASE + wid, 1)],
                        dstbuf.at[pl.ds(0, 1)])

        def xbody(j, carry):
            count_row(0, j)
            return carry

        lax.fori_loop(0, gpr, xbody, 0)

    pltpu.sync_copy(histbuf, hist_out.at[wid])


def _deg_call(ei3):
    mesh = plsc.VectorSubcoreMesh(core_axis_name="c", subcore_axis_name="s")
    return pl.kernel(
        _deg_body,
        out_type=jax.ShapeDtypeStruct((NW, N_NODES // L, L), jnp.float32),
        mesh=mesh,
        scratch_types=[
            pltpu.VMEM((RPT, IDX_W), jnp.int32),
            pltpu.VMEM((N_NODES // L, L), jnp.float32),
        ],
        compiler_params=pltpu.CompilerParams(
            needs_layout_passes=False, use_tc_tiling_on_sc=False),
    )(ei3)


RPS = 320
R16 = RPS // L


def _rsqrt_vec(deg):
    i = plsc.bitcast(deg, jnp.int32)
    y = plsc.bitcast(jnp.int32(0x5F3759DF) - (i >> 1), jnp.float32)
    for _ in range(3):
        y = y * (1.5 - 0.5 * deg * y * y)
    return y


def _g0sc_body(z_hbm, hist_hbm, g0_hbm, dinv_hbm, d2_hbm,
               zv, hv, g0v, dv, d2v, sem):
    c = lax.axis_index("c")
    s = lax.axis_index("s")
    wid = s * NC + c
    base = jnp.minimum(wid * RPS, N_NODES - RPS)
    b16 = jnp.minimum(wid * R16, N_NODES // L - R16)
    cp1 = pltpu.async_copy(hist_hbm.at[:, pl.ds(b16, R16), :], hv, sem)
    cp2 = pltpu.async_copy(z_hbm.at[pl.ds(base, RPS)], zv, sem)
    cp1.wait()
    cp2.wait()

    def dbody(r2, carry):
        acc = hv[0, r2, pl.ds(0, L)]
        for w in range(1, NW):
            acc = acc + hv[w, r2, pl.ds(0, L)]
        deg = acc + 1.0
        y = _rsqrt_vec(deg)
        dv[pl.ds(r2 * L, L)] = y
        d2v[pl.ds(r2 * L, L)] = y * y
        return carry

    lax.fori_loop(0, R16, dbody, 0)

    def rbody(r4, carry):
        for u in range(4):
            r = r4 * 4 + u
            dd = plsc.load_gather(dv, [jnp.full((L,), r, jnp.int32)])
            for j in range(C_PAD // L):
                g0v[r, pl.ds(j * L, L)] = zv[r, pl.ds(j * L, L)] * dd
        return carry

    lax.fori_loop(0, RPS // 4, rbody, 0)
    pltpu.sync_copy(g0v, g0_hbm.at[pl.ds(base, RPS)])
    pltpu.sync_copy(dv, dinv_hbm.at[pl.ds(base, RPS)])
    pltpu.sync_copy(d2v, d2_hbm.at[pl.ds(base, RPS)])


def _g0sc_call(z, hist3):
    mesh = plsc.VectorSubcoreMesh(core_axis_name="c", subcore_axis_name="s")
    return pl.kernel(
        _g0sc_body,
        out_type=[
            jax.ShapeDtypeStruct((N_NODES, C_PAD), jnp.float32),
            jax.ShapeDtypeStruct((N_NODES,), jnp.float32),
            jax.ShapeDtypeStruct((N_NODES,), jnp.float32),
        ],
        mesh=mesh,
        scratch_types=[
            pltpu.VMEM((RPS, C_PAD), jnp.float32),
            pltpu.VMEM((NW, R16, L), jnp.float32),
            pltpu.VMEM((RPS, C_PAD), jnp.float32),
            pltpu.VMEM((RPS,), jnp.float32),
            pltpu.VMEM((RPS,), jnp.float32),
            pltpu.SemaphoreType.DMA,
        ],
        compiler_params=pltpu.CompilerParams(
            needs_layout_passes=False, use_tc_tiling_on_sc=False),
    )(z, hist3)


def _scalesc_body(g0_hbm, parts_hbm, d2_hbm, g1_hbm,
                  g0v, pav, pbv, d2v, g1v, sem):
    c = lax.axis_index("c")
    s = lax.axis_index("s")
    wid = s * NC + c
    base = jnp.minimum(wid * RPS, N_NODES - RPS)
    cps = [
        pltpu.async_copy(g0_hbm.at[pl.ds(base, RPS)], g0v, sem),
        pltpu.async_copy(parts_hbm.at[0, pl.ds(base, RPS)], pav, sem),
        pltpu.async_copy(parts_hbm.at[1, pl.ds(base, RPS)], pbv, sem),
        pltpu.async_copy(d2_hbm.at[pl.ds(base, RPS)], d2v, sem),
    ]
    for cp in cps:
        cp.wait()

    def rbody(r4, carry):
        for u in range(4):
            r = r4 * 4 + u
            dd = plsc.load_gather(d2v, [jnp.full((L,), r, jnp.int32)])
            for j in range(C_PAD // L):
                sl = pl.ds(j * L, L)
                g1v[r, sl] = (g0v[r, sl] + pav[r, sl] + pbv[r, sl]) * dd
        return carry

    lax.fori_loop(0, RPS // 4, rbody, 0)
    pltpu.sync_copy(g1v, g1_hbm.at[pl.ds(base, RPS)])


def _scalesc_call(g0, parts, d2):
    mesh = plsc.VectorSubcoreMesh(core_axis_name="c", subcore_axis_name="s")
    return pl.kernel(
        _scalesc_body,
        out_type=jax.ShapeDtypeStruct((N_NODES, C_PAD), jnp.float32),
        mesh=mesh,
        scratch_types=[
            pltpu.VMEM((RPS, C_PAD), jnp.float32),
            pltpu.VMEM((RPS, C_PAD), jnp.float32),
            pltpu.VMEM((RPS, C_PAD), jnp.float32),
            pltpu.VMEM((RPS,), jnp.float32),
            pltpu.VMEM((RPS, C_PAD), jnp.float32),
            pltpu.SemaphoreType.DMA,
        ],
        compiler_params=pltpu.CompilerParams(
            needs_layout_passes=False, use_tc_tiling_on_sc=False),
    )(g0, parts, d2)


def _prop_body(g_hbm, ei_hbm, out_hbm, srcbuf, dstbuf, *rest):
    bufs_a = rest[:G]
    bufs_b = rest[G:2 * G]
    semga, semgb, semsa, semsb, acc = rest[2 * G:]
    c = lax.axis_index("c")
    s = lax.axis_index("s")
    wid = s * NC + c
    rps = N_NODES // NS

    with jax.named_scope("ldidx"):
        cps = pltpu.async_copy(ei_hbm.at[0, pl.ds(wid * RPT, RPT)], srcbuf,
                               semga)
        cpd = pltpu.async_copy(ei_hbm.at[1, pl.ds(wid * RPT, RPT)], dstbuf,
                               semgb)
    zeros = jnp.zeros((L,), jnp.float32)

    def zbody(i, carry):
        r = i // (C_PAD // L)
        j = i % (C_PAD // L)
        bufs_a[0][r, pl.ds(j * L, L)] = zeros
        return carry

    lax.fori_loop(0, IDX_W * (C_PAD // L), zbody, 0)

    zch = rps // 5

    def zcopy(i, carry):
        pltpu.sync_copy(bufs_a[0].at[pl.ds(0, zch)],
                        acc.at[pl.ds(s * rps + i * zch, zch)])
        return carry

    with jax.named_scope("acc_zero"):
        lax.fori_loop(0, 5, zcopy, 0)
    cps.wait()
    cpd.wait()

    def gath(j, buf, sem):
        pltpu.async_copy(g_hbm.at[srcbuf.at[j]], buf, sem)

    def gwait(buf, sem):
        pltpu.make_async_copy(g_hbm.at[srcbuf.at[0]], buf, sem).wait()

    def scat(j, buf, sem):
        pltpu.async_copy(buf, acc.at[dstbuf.at[j]], sem, add=True)

    def swait(buf, sem):
        pltpu.make_async_copy(buf, acc.at[dstbuf.at[0]], sem).wait()

    for k in range(G):
        gath(k, bufs_a[k], semga)
    plsc.subcore_barrier()

    def pbody(i, carry):
        ja = (2 * i) * G
        jb = ja + G
        for k in range(G):
            gath(jb + k, bufs_b[k], semgb)
        for k in range(G):
            gwait(bufs_a[k], semga)
        for k in range(G):
            scat(ja + k, bufs_a[k], semsa)

        @pl.when(i < NPAIR - 1)
        def _():
            for k in range(G):
                swait(bufs_a[k], semsa)
            for k in range(G):
                gath(jb + G + k, bufs_a[k], semga)

        for k in range(G):
            gwait(bufs_b[k], semgb)
        for k in range(G):
            scat(jb + k, bufs_b[k], semsb)
        for k in range(G):
            swait(bufs_b[k], semsb)
        return carry

    with jax.named_scope("edges"):
        lax.fori_loop(0, NPAIR, pbody, 0)
        for k in range(G):
            swait(bufs_a[k], semsa)

        @pl.when(wid < N_XTRA)
        def _():
            pltpu.sync_copy(ei_hbm.at[0, pl.ds(XBASE + wid, 1)],
                            srcbuf.at[pl.ds(0, 1)])
            pltpu.sync_copy(ei_hbm.at[1, pl.ds(XBASE + wid, 1)],
                            dstbuf.at[pl.ds(0, 1)])
            gath(0, bufs_a[0], semga)
            gwait(bufs_a[0], semga)
            scat(0, bufs_a[0], semsa)
            swait(bufs_a[0], semsa)

    plsc.subcore_barrier()
    with jax.named_scope("wb"):
        pltpu.sync_copy(acc.at[pl.ds(s * rps, rps)],
                        out_hbm.at[c, pl.ds(s * rps, rps)])


def _prop_call(g, ei3):
    mesh = plsc.VectorSubcoreMesh(core_axis_name="c", subcore_axis_name="s")
    return pl.kernel(
        _prop_body,
        out_type=jax.ShapeDtypeStruct((NC, N_NODES, C_PAD), jnp.float32),
        mesh=mesh,
        scratch_types=(
            [pltpu.VMEM((RPT, IDX_W), jnp.int32),
             pltpu.VMEM((RPT, IDX_W), jnp.int32)]
            + [pltpu.VMEM((IDX_W, C_PAD), jnp.float32)] * (2 * G)
            + [pltpu.SemaphoreType.DMA] * 4
            + [pltpu.VMEM_SHARED((N_NODES, C_PAD), jnp.float32)]
        ),
        compiler_params=pltpu.CompilerParams(
            needs_layout_passes=False, use_tc_tiling_on_sc=False),
    )(g, ei3)


def _mm_body(x_ref, w_ref, z_ref):
    z_ref[...] = jnp.dot(x_ref[...], w_ref[...],
                         preferred_element_type=jnp.float32)


def _mm_call(x, w_pad):
    return pl.pallas_call(
        _mm_body,
        grid=(N_NODES // BLK,),
        in_specs=[
            pl.BlockSpec((BLK, D_FEAT), lambda i: (i, 0)),
            pl.BlockSpec((D_FEAT, C_PAD), lambda i: (0, 0)),
        ],
        out_specs=pl.BlockSpec((BLK, C_PAD), lambda i: (i, 0)),
        out_shape=jax.ShapeDtypeStruct((N_NODES, C_PAD), jnp.float32),
    )(x, w_pad)


def _final_body(g1_ref, q_ref, dinv_ref, b_ref, out_ref):
    t = g1_ref[...] + q_ref[0] + q_ref[1]
    dinv = dinv_ref[pl.program_id(0)]
    logits = (t * dinv[:, None])[:, :C_OUT] + b_ref[...][None, :]
    m = jnp.max(logits, axis=1, keepdims=True)
    e = jnp.exp(logits - m)
    out_ref[...] = logits - m - jnp.log(jnp.sum(e, axis=1, keepdims=True))


def _final_call(g1, parts, dinv, b):
    return pl.pallas_call(
        _final_body,
        grid=(N_NODES // BLK,),
        in_specs=[
            pl.BlockSpec((BLK, C_PAD), lambda i: (i, 0)),
            pl.BlockSpec((NC, BLK, C_PAD), lambda i: (0, i, 0)),
            pl.BlockSpec((N_NODES // BLK, BLK), lambda i: (0, 0)),
            pl.BlockSpec((C_OUT,), lambda i: (0,)),
        ],
        out_specs=pl.BlockSpec((BLK, C_OUT), lambda i: (i, 0)),
        out_shape=jax.ShapeDtypeStruct((N_NODES, C_OUT), jnp.float32),
    )(g1, parts, dinv.reshape(N_NODES // BLK, BLK), b)


def kernel(x, edge_index, W, b):
    ei3 = edge_index.astype(jnp.int32).reshape(2, ROWS_E, IDX_W)
    w_pad = jnp.pad(W, ((0, 0), (0, C_PAD - C_OUT)))

    z = _mm_call(x, w_pad)
    hist3 = _deg_call(ei3)
    g0, dinv, d2 = _g0sc_call(z, hist3)
    s0 = _prop_call(g0, ei3)
    g1 = _scalesc_call(g0, s0, d2)
    s1 = _prop_call(g1, ei3)
    return _final_call(g1, s1, dinv, b)


# speedup vs baseline: 1.1493x; 1.0313x over previous
import jax
import jax.numpy as jnp
from jax import lax
from jax.experimental import pallas as pl
from jax.experimental.pallas import tpu as pltpu
from jax.experimental.pallas import tpu_sc as plsc

N_NODES = 10000
D_FEAT = 128
C_OUT = 40
C_PAD = 40
COLS = (0, 16, 24)
E_ORIG = 320000

NC, NS, L = 2, 16, 16
NW = NC * NS

IDX_W = 128
ROWS_E = E_ORIG // IDX_W
RPT = ROWS_E // NW
N_XTRA = ROWS_E - RPT * NW
XBASE = RPT * NW
BLK = 1000
G = 3
NPAIR = RPT // (2 * G)


def _deg_body(ei_hbm, hist_out, dstbuf, histbuf):
    c = lax.axis_index("c")
    s = lax.axis_index("s")
    wid = s * NC + c
    zeros = jnp.zeros((L,), jnp.float32)

    def zbody(i, carry):
        histbuf[i, pl.ds(0, L)] = zeros
        return carry

    lax.fori_loop(0, N_NODES // L, zbody, 0)
    pltpu.sync_copy(ei_hbm.at[1, pl.ds(wid * RPT, RPT)], dstbuf)
    ones = jnp.ones((L,), jnp.float32)
    gpr = IDX_W // L

    def count_row(r, j):
        idx = dstbuf[r, pl.ds(j * L, L)]
        plsc.addupdate_scatter(histbuf, [idx >> 4, idx & 15], ones)

    def body(i, carry):
        count_row(i // gpr, i % gpr)
        return carry

    lax.fori_loop(0, RPT * gpr, body, 0)

    @pl.when(wid < N_XTRA)
    def _():
        pltpu.sync_copy(ei_hbm.at[1, pl.ds(XBASE + wid, 1)],
                        dstbuf.at[pl.ds(0, 1)])

        def xbody(j, carry):
            count_row(0, j)
            return carry

        lax.fori_loop(0, gpr, xbody, 0)

    pltpu.sync_copy(histbuf, hist_out.at[wid])


def _deg_call(ei3):
    mesh = plsc.VectorSubcoreMesh(core_axis_name="c", subcore_axis_name="s")
    return pl.kernel(
        _deg_body,
        out_type=jax.ShapeDtypeStruct((NW, N_NODES // L, L), jnp.float32),
        mesh=mesh,
        scratch_types=[
            pltpu.VMEM((RPT, IDX_W), jnp.int32),
            pltpu.VMEM((N_NODES // L, L), jnp.float32),
        ],
        compiler_params=pltpu.CompilerParams(
            needs_layout_passes=False, use_tc_tiling_on_sc=False),
    )(ei3)


RPS = 320
R16 = RPS // L


def _rsqrt_vec(deg):
    i = plsc.bitcast(deg, jnp.int32)
    y = plsc.bitcast(jnp.int32(0x5F3759DF) - (i >> 1), jnp.float32)
    for _ in range(3):
        y = y * (1.5 - 0.5 * deg * y * y)
    return y


def _g0sc_body(z_hbm, hist_hbm, g0_hbm, dinv_hbm, d2_hbm,
               zv, hv, g0v, dv, d2v, sem):
    c = lax.axis_index("c")
    s = lax.axis_index("s")
    wid = s * NC + c
    base = jnp.minimum(wid * RPS, N_NODES - RPS)
    b16 = jnp.minimum(wid * R16, N_NODES // L - R16)
    cp1 = pltpu.async_copy(hist_hbm.at[:, pl.ds(b16, R16), :], hv, sem)
    cp2 = pltpu.async_copy(z_hbm.at[pl.ds(base, RPS)], zv, sem)
    cp1.wait()
    cp2.wait()

    def dbody(r2, carry):
        acc = hv[0, r2, pl.ds(0, L)]
        for w in range(1, NW):
            acc = acc + hv[w, r2, pl.ds(0, L)]
        deg = acc + 1.0
        y = _rsqrt_vec(deg)
        dv[pl.ds(r2 * L, L)] = y
        d2v[pl.ds(r2 * L, L)] = y * y
        return carry

    lax.fori_loop(0, R16, dbody, 0)

    def rbody(r4, carry):
        for u in range(4):
            r = r4 * 4 + u
            dd = plsc.load_gather(dv, [jnp.full((L,), r, jnp.int32)])
            for o in COLS:
                g0v[r, pl.ds(o, L)] = zv[r, pl.ds(o, L)] * dd
        return carry

    lax.fori_loop(0, RPS // 4, rbody, 0)
    pltpu.sync_copy(g0v, g0_hbm.at[pl.ds(base, RPS)])
    pltpu.sync_copy(dv, dinv_hbm.at[pl.ds(base, RPS)])
    pltpu.sync_copy(d2v, d2_hbm.at[pl.ds(base, RPS)])


def _g0sc_call(z, hist3):
    mesh = plsc.VectorSubcoreMesh(core_axis_name="c", subcore_axis_name="s")
    return pl.kernel(
        _g0sc_body,
        out_type=[
            jax.ShapeDtypeStruct((N_NODES, C_PAD), jnp.float32),
            jax.ShapeDtypeStruct((N_NODES,), jnp.float32),
            jax.ShapeDtypeStruct((N_NODES,), jnp.float32),
        ],
        mesh=mesh,
        scratch_types=[
            pltpu.VMEM((RPS, C_PAD), jnp.float32),
            pltpu.VMEM((NW, R16, L), jnp.float32),
            pltpu.VMEM((RPS, C_PAD), jnp.float32),
            pltpu.VMEM((RPS,), jnp.float32),
            pltpu.VMEM((RPS,), jnp.float32),
            pltpu.SemaphoreType.DMA,
        ],
        compiler_params=pltpu.CompilerParams(
            needs_layout_passes=False, use_tc_tiling_on_sc=False),
    )(z, hist3)


def _scalesc_body(g0_hbm, parts_hbm, d2_hbm, g1_hbm,
                  g0v, pav, pbv, d2v, g1v, sem):
    c = lax.axis_index("c")
    s = lax.axis_index("s")
    wid = s * NC + c
    base = jnp.minimum(wid * RPS, N_NODES - RPS)
    cps = [
        pltpu.async_copy(g0_hbm.at[pl.ds(base, RPS)], g0v, sem),
        pltpu.async_copy(parts_hbm.at[0, pl.ds(base, RPS)], pav, sem),
        pltpu.async_copy(parts_hbm.at[1, pl.ds(base, RPS)], pbv, sem),
        pltpu.async_copy(d2_hbm.at[pl.ds(base, RPS)], d2v, sem),
    ]
    for cp in cps:
        cp.wait()

    def rbody(r4, carry):
        for u in range(4):
            r = r4 * 4 + u
            dd = plsc.load_gather(d2v, [jnp.full((L,), r, jnp.int32)])
            for o in COLS:
                sl = pl.ds(o, L)
                g1v[r, sl] = (g0v[r, sl] + pav[r, sl] + pbv[r, sl]) * dd
        return carry

    lax.fori_loop(0, RPS // 4, rbody, 0)
    pltpu.sync_copy(g1v, g1_hbm.at[pl.ds(base, RPS)])


def _scalesc_call(g0, parts, d2):
    mesh = plsc.VectorSubcoreMesh(core_axis_name="c", subcore_axis_name="s")
    return pl.kernel(
        _scalesc_body,
        out_type=jax.ShapeDtypeStruct((N_NODES, C_PAD), jnp.float32),
        mesh=mesh,
        scratch_types=[
            pltpu.VMEM((RPS, C_PAD), jnp.float32),
            pltpu.VMEM((RPS, C_PAD), jnp.float32),
            pltpu.VMEM((RPS, C_PAD), jnp.float32),
            pltpu.VMEM((RPS,), jnp.float32),
            pltpu.VMEM((RPS, C_PAD), jnp.float32),
            pltpu.SemaphoreType.DMA,
        ],
        compiler_params=pltpu.CompilerParams(
            needs_layout_passes=False, use_tc_tiling_on_sc=False),
    )(g0, parts, d2)


def _prop_body(g_hbm, ei_hbm, out_hbm, srcbuf, dstbuf, *rest):
    bufs_a = rest[:G]
    bufs_b = rest[G:2 * G]
    semga, semgb, semsa, semsb, acc = rest[2 * G:]
    c = lax.axis_index("c")
    s = lax.axis_index("s")
    wid = s * NC + c
    rps = N_NODES // NS

    with jax.named_scope("ldidx"):
        cps = pltpu.async_copy(ei_hbm.at[0, pl.ds(wid * RPT, RPT)], srcbuf,
                               semga)
        cpd = pltpu.async_copy(ei_hbm.at[1, pl.ds(wid * RPT, RPT)], dstbuf,
                               semgb)
    zeros = jnp.zeros((L,), jnp.float32)

    def zbody(r, carry):
        for o in COLS:
            bufs_a[0][r, pl.ds(o, L)] = zeros
        return carry

    lax.fori_loop(0, IDX_W, zbody, 0)

    zch = rps // 5

    def zcopy(i, carry):
        pltpu.sync_copy(bufs_a[0].at[pl.ds(0, zch)],
                        acc.at[pl.ds(s * rps + i * zch, zch)])
        return carry

    with jax.named_scope("acc_zero"):
        lax.fori_loop(0, 5, zcopy, 0)
    cps.wait()
    cpd.wait()

    def gath(j, buf, sem):
        pltpu.async_copy(g_hbm.at[srcbuf.at[j]], buf, sem)

    def gwait(buf, sem):
        pltpu.make_async_copy(g_hbm.at[srcbuf.at[0]], buf, sem).wait()

    def scat(j, buf, sem):
        pltpu.async_copy(buf, acc.at[dstbuf.at[j]], sem, add=True)

    def swait(buf, sem):
        pltpu.make_async_copy(buf, acc.at[dstbuf.at[0]], sem).wait()

    for k in range(G):
        gath(k, bufs_a[k], semga)
    plsc.subcore_barrier()

    def pbody(i, carry):
        ja = (2 * i) * G
        jb = ja + G
        for k in range(G):
            gath(jb + k, bufs_b[k], semgb)
        for k in range(G):
            gwait(bufs_a[k], semga)
        for k in range(G):
            scat(ja + k, bufs_a[k], semsa)

        @pl.when(i < NPAIR - 1)
        def _():
            for k in range(G):
                swait(bufs_a[k], semsa)
            for k in range(G):
                gath(jb + G + k, bufs_a[k], semga)

        for k in range(G):
            gwait(bufs_b[k], semgb)
        for k in range(G):
            scat(jb + k, bufs_b[k], semsb)
        for k in range(G):
            swait(bufs_b[k], semsb)
        return carry

    with jax.named_scope("edges"):
        lax.fori_loop(0, NPAIR, pbody, 0)
        for k in range(G):
            swait(bufs_a[k], semsa)

        @pl.when(wid < N_XTRA)
        def _():
            pltpu.sync_copy(ei_hbm.at[0, pl.ds(XBASE + wid, 1)],
                            srcbuf.at[pl.ds(0, 1)])
            pltpu.sync_copy(ei_hbm.at[1, pl.ds(XBASE + wid, 1)],
                            dstbuf.at[pl.ds(0, 1)])
            gath(0, bufs_a[0], semga)
            gwait(bufs_a[0], semga)
            scat(0, bufs_a[0], semsa)
            swait(bufs_a[0], semsa)

    plsc.subcore_barrier()
    with jax.named_scope("wb"):
        pltpu.sync_copy(acc.at[pl.ds(s * rps, rps)],
                        out_hbm.at[c, pl.ds(s * rps, rps)])


def _prop_call(g, ei3):
    mesh = plsc.VectorSubcoreMesh(core_axis_name="c", subcore_axis_name="s")
    return pl.kernel(
        _prop_body,
        out_type=jax.ShapeDtypeStruct((NC, N_NODES, C_PAD), jnp.float32),
        mesh=mesh,
        scratch_types=(
            [pltpu.VMEM((RPT, IDX_W), jnp.int32),
             pltpu.VMEM((RPT, IDX_W), jnp.int32)]
            + [pltpu.VMEM((IDX_W, C_PAD), jnp.float32)] * (2 * G)
            + [pltpu.SemaphoreType.DMA] * 4
            + [pltpu.VMEM_SHARED((N_NODES, C_PAD), jnp.float32)]
        ),
        compiler_params=pltpu.CompilerParams(
            needs_layout_passes=False, use_tc_tiling_on_sc=False),
    )(g, ei3)


def _mm_body(x_ref, w_ref, z_ref):
    z_ref[...] = jnp.dot(x_ref[...], w_ref[...],
                         preferred_element_type=jnp.float32)


def _mm_call(x, w):
    return pl.pallas_call(
        _mm_body,
        grid=(N_NODES // BLK,),
        in_specs=[
            pl.BlockSpec((BLK, D_FEAT), lambda i: (i, 0)),
            pl.BlockSpec((D_FEAT, C_PAD), lambda i: (0, 0)),
        ],
        out_specs=pl.BlockSpec((BLK, C_PAD), lambda i: (i, 0)),
        out_shape=jax.ShapeDtypeStruct((N_NODES, C_PAD), jnp.float32),
    )(x, w)


def _final_body(g1_ref, q_ref, dinv_ref, b_ref, out_ref):
    t = g1_ref[...] + q_ref[0] + q_ref[1]
    dinv = dinv_ref[pl.program_id(0)]
    logits = t * dinv[:, None] + b_ref[...][None, :]
    m = jnp.max(logits, axis=1, keepdims=True)
    e = jnp.exp(logits - m)
    out_ref[...] = logits - m - jnp.log(jnp.sum(e, axis=1, keepdims=True))


def _final_call(g1, parts, dinv, b):
    return pl.pallas_call(
        _final_body,
        grid=(N_NODES // BLK,),
        in_specs=[
            pl.BlockSpec((BLK, C_PAD), lambda i: (i, 0)),
            pl.BlockSpec((NC, BLK, C_PAD), lambda i: (0, i, 0)),
            pl.BlockSpec((N_NODES // BLK, BLK), lambda i: (0, 0)),
            pl.BlockSpec((C_OUT,), lambda i: (0,)),
        ],
        out_specs=pl.BlockSpec((BLK, C_OUT), lambda i: (i, 0)),
        out_shape=jax.ShapeDtypeStruct((N_NODES, C_OUT), jnp.float32),
    )(g1, parts, dinv.reshape(N_NODES // BLK, BLK), b)


def kernel(x, edge_index, W, b):
    ei3 = edge_index.astype(jnp.int32).reshape(2, ROWS_E, IDX_W)
    z = _mm_call(x, W)
    hist3 = _deg_call(ei3)
    g0, dinv, d2 = _g0sc_call(z, hist3)
    s0 = _prop_call(g0, ei3)
    g1 = _scalesc_call(g0, s0, d2)
    s1 = _prop_call(g1, ei3)
    return _final_call(g1, s1, dinv, b)
